# baseline (speedup 1.0000x reference)
import jax
import jax.numpy as jnp
from jax.experimental import pallas as pl
from jax.experimental.pallas import tpu as pltpu

MAX_SAMPLES_ = 1000000
N_DIM_ = 64
BATCH_ = 16384

WORDS = (MAX_SAMPLES_ * N_DIM_) // 2
NV = WORDS // 128
AW = (BATCH_ * N_DIM_) // 2
NA = AW // 128 + 1
BW = 2000
GRID = NV // BW
NA2P = ((NA + 2 * BW + 7) // 8) * 8 + 8


def _merge_body(scal_ref, a2p_ref, cache_ref, out_ref):
    g = pl.program_id(0)
    r0 = scal_ref[0]
    c0 = scal_ref[1]
    s = g * BW

    shift = s - r0
    shift = jnp.where(shift < 0, shift + NV, shift)
    shift2 = jnp.where(shift >= NV - BW, shift - NV, shift)
    start = jnp.clip(shift2 + BW, 0, NA2P - BW - 8)

    start8 = pl.multiple_of((start // 8) * 8, 8)
    rem = start - start8
    w8 = a2p_ref[pl.ds(start8, BW + 8), :]
    w = pltpu.roll(w8, (BW + 8) - rem, axis=0)[:BW, :]

    rr = jax.lax.broadcasted_iota(jnp.int32, (BW, 128), 0)
    ll = jax.lax.broadcasted_iota(jnp.int32, (BW, 128), 1)
    rowpos = shift2 + rr
    pos_w = rowpos * 128 + ll - c0
    mask = (pos_w >= 0) & (pos_w < AW)

    out_ref[...] = jnp.where(mask, w, cache_ref[...])


def kernel(activations, cache, n_valid, index):
    max_samples = cache.shape[0]
    batch = activations.shape[0]
    index = jnp.asarray(index) % max_samples
    new_n_valid = jnp.minimum(jnp.asarray(n_valid) + batch, max_samples)
    new_index = (index + batch) % max_samples

    w0 = index * (N_DIM_ // 2)
    r0 = (w0 // 128).astype(jnp.int32)
    c0 = (w0 % 128).astype(jnp.int32)
    scal = jnp.stack([r0, c0])

    acts_w = jax.lax.bitcast_convert_type(
        activations.astype(jnp.float16).reshape(AW, 2), jnp.int32)
    zhead = jnp.zeros((96,), jnp.int32)
    ztail = jnp.zeros((224,), jnp.int32)
    padded = jnp.concatenate([zhead, acts_w, ztail])
    a2 = jax.lax.dynamic_slice(padded, (96 - c0,), (NA * 128,)).reshape(NA, 128)
    a2p = jnp.concatenate(
        [jnp.zeros((BW, 128), jnp.int32), a2,
         jnp.zeros((NA2P - BW - NA, 128), jnp.int32)], axis=0)

    cache_w = jax.lax.bitcast_convert_type(
        cache.reshape(WORDS, 2), jnp.int32).reshape(NV, 128)

    out_w = pl.pallas_call(
        _merge_body,
        grid=(GRID,),
        in_specs=[
            pl.BlockSpec(memory_space=pltpu.SMEM),
            pl.BlockSpec((NA2P, 128), lambda g: (0, 0)),
            pl.BlockSpec((BW, 128), lambda g: (g, 0)),
        ],
        out_specs=pl.BlockSpec((BW, 128), lambda g: (g, 0)),
        out_shape=jax.ShapeDtypeStruct((NV, 128), jnp.int32),
    )(scal, a2p, cache_w)

    new_cache = jax.lax.bitcast_convert_type(
        out_w.reshape(WORDS), jnp.float16).reshape(max_samples, N_DIM_)
    return (new_cache, new_n_valid, new_index)


# speedup vs baseline: 1.0012x; 1.0012x over previous
import jax
import jax.numpy as jnp
from jax.experimental import pallas as pl
from jax.experimental.pallas import tpu as pltpu

MAX_SAMPLES_ = 1000000
N_DIM_ = 64
BATCH_ = 16384

WORDS = (MAX_SAMPLES_ * N_DIM_) // 2
NV = WORDS // 128
AW = (BATCH_ * N_DIM_) // 2
NA = AW // 128 + 1
BW = 2000
GRID = NV // BW
NA2P = ((NA + 2 * BW + 7) // 8) * 8 + 8


def _merge_body(scal_ref, a2p_ref, cache_ref, out_ref):
    g = pl.program_id(0)
    r0 = scal_ref[0]
    c0 = scal_ref[1]
    s = g * BW

    shift = s - r0
    shift = jnp.where(shift < 0, shift + NV, shift)
    shift2 = jnp.where(shift >= NV - BW, shift - NV, shift)
    overlap = shift2 < NA

    @pl.when(jnp.logical_not(overlap))
    def _copy():
        out_ref[...] = cache_ref[...]

    @pl.when(overlap)
    def _merge():
        start = jnp.clip(shift2 + BW, 0, NA2P - BW - 8)
        start8 = pl.multiple_of((start // 8) * 8, 8)
        rem = start - start8
        w8 = a2p_ref[pl.ds(start8, BW + 8), :]
        w = pltpu.roll(w8, (BW + 8) - rem, axis=0)[:BW, :]

        rr = jax.lax.broadcasted_iota(jnp.int32, (BW, 128), 0)
        ll = jax.lax.broadcasted_iota(jnp.int32, (BW, 128), 1)
        rowpos = shift2 + rr
        pos_w = rowpos * 128 + ll - c0
        mask = (pos_w >= 0) & (pos_w < AW)

        out_ref[...] = jnp.where(mask, w, cache_ref[...])


def kernel(activations, cache, n_valid, index):
    max_samples = cache.shape[0]
    batch = activations.shape[0]
    index = jnp.asarray(index) % max_samples
    new_n_valid = jnp.minimum(jnp.asarray(n_valid) + batch, max_samples)
    new_index = (index + batch) % max_samples

    w0 = index * (N_DIM_ // 2)
    r0 = (w0 // 128).astype(jnp.int32)
    c0 = (w0 % 128).astype(jnp.int32)
    scal = jnp.stack([r0, c0])

    acts_w = jax.lax.bitcast_convert_type(
        activations.astype(jnp.float16).reshape(AW, 2), jnp.int32)
    zhead = jnp.zeros((96,), jnp.int32)
    ztail = jnp.zeros((224,), jnp.int32)
    padded = jnp.concatenate([zhead, acts_w, ztail])
    a2 = jax.lax.dynamic_slice(padded, (96 - c0,), (NA * 128,)).reshape(NA, 128)
    a2p = jnp.concatenate(
        [jnp.zeros((BW, 128), jnp.int32), a2,
         jnp.zeros((NA2P - BW - NA, 128), jnp.int32)], axis=0)

    cache_w = jax.lax.bitcast_convert_type(
        cache.reshape(WORDS, 2), jnp.int32).reshape(NV, 128)

    out_w = pl.pallas_call(
        _merge_body,
        grid=(GRID,),
        in_specs=[
            pl.BlockSpec(memory_space=pltpu.SMEM),
            pl.BlockSpec((NA2P, 128), lambda g: (0, 0)),
            pl.BlockSpec((BW, 128), lambda g: (g, 0)),
        ],
        out_specs=pl.BlockSpec((BW, 128), lambda g: (g, 0)),
        out_shape=jax.ShapeDtypeStruct((NV, 128), jnp.int32),
    )(scal, a2p, cache_w)

    new_cache = jax.lax.bitcast_convert_type(
        out_w.reshape(WORDS), jnp.float16).reshape(max_samples, N_DIM_)
    return (new_cache, new_n_valid, new_index)


# speedup vs baseline: 11.5564x; 11.5428x over previous
import jax
import jax.numpy as jnp
from jax.experimental import pallas as pl
from jax.experimental.pallas import tpu as pltpu

MAX_SAMPLES_ = 1000000
N_DIM_ = 64
BATCH_ = 16384
NW = N_DIM_ // 2

BC = 8192
NB = -(-MAX_SAMPLES_ // BC)
G2 = (BC - 1 + BATCH_ + BC - 1) // BC
OVC = 2 * G2 * BC


def _body(s_ref, ovl_ref, cache_ref, out_ref):
    g = pl.program_id(0)
    idx = s_ref[1]
    e1 = s_ref[2]
    e2 = s_ref[3]

    near = ((g >= s_ref[0]) & (g < s_ref[0] + G2)) | (g < G2)

    @pl.when(jnp.logical_not(near))
    def _copy():
        out_ref[...] = cache_ref[...]

    @pl.when(near)
    def _merge():
        c = g * BC + jax.lax.broadcasted_iota(jnp.int32, (NW, BC), 1)
        mask = ((c >= idx) & (c < e1)) | (c < e2)
        out_ref[...] = jnp.where(mask, ovl_ref[...], cache_ref[...])


def kernel(activations, cache, n_valid, index):
    max_samples = cache.shape[0]
    batch = activations.shape[0]
    index = jnp.asarray(index) % max_samples
    new_n_valid = jnp.minimum(jnp.asarray(n_valid) + batch, max_samples)
    new_index = (index + batch) % max_samples

    q1 = (index // BC).astype(jnp.int32)
    im1 = (index % BC).astype(jnp.int32)
    e1 = jnp.minimum(index + batch, max_samples).astype(jnp.int32)
    e2 = (index + batch - max_samples).astype(jnp.int32)
    sp = jnp.stack([q1, index.astype(jnp.int32), e1, e2])

    cache_wt = jax.lax.bitcast_convert_type(
        cache.reshape(max_samples, NW, 2), jnp.int32).T
    acts_wt = jax.lax.bitcast_convert_type(
        activations.astype(jnp.float16).reshape(batch, NW, 2), jnp.int32).T

    ovl1 = jax.lax.dynamic_update_slice(
        jnp.zeros((NW, G2 * BC), jnp.int32), acts_wt, (0, im1))
    delta = jnp.where(e2 > 0, max_samples - index, batch)
    ovl2 = jax.lax.dynamic_slice(
        jnp.concatenate(
            [acts_wt, jnp.zeros((NW, G2 * BC), jnp.int32)], axis=1),
        (0, delta), (NW, G2 * BC))
    ovl = jnp.concatenate([ovl1, ovl2], axis=1)

    def ovl_map(g, s):
        in_w1 = (g >= s[0]) & (g < s[0] + G2)
        return (0, jnp.where(in_w1, g - s[0],
                             jnp.where(g < G2, G2 + g, 0)))

    grid_spec = pltpu.PrefetchScalarGridSpec(
        num_scalar_prefetch=1,
        grid=(NB,),
        in_specs=[
            pl.BlockSpec((NW, BC), ovl_map),
            pl.BlockSpec((NW, BC), lambda g, s: (0, g)),
        ],
        out_specs=pl.BlockSpec((NW, BC), lambda g, s: (0, g)),
    )

    out_wt = pl.pallas_call(
        _body,
        grid_spec=grid_spec,
        out_shape=jax.ShapeDtypeStruct((NW, max_samples), jnp.int32),
    )(sp, ovl, cache_wt)

    new_cache = jax.lax.bitcast_convert_type(
        out_wt.T, jnp.float16).reshape(max_samples, N_DIM_)
    return (new_cache, new_n_valid, new_index)


# speedup vs baseline: 62.9386x; 5.4462x over previous
import jax
import jax.numpy as jnp
from jax.experimental import pallas as pl
from jax.experimental.pallas import tpu as pltpu

MAX_SAMPLES_ = 1000000
N_DIM_ = 64
BATCH_ = 16384

BF = 16
NBF = N_DIM_ // BF
BC = 16384
NBC = -(-MAX_SAMPLES_ // BC)
G2 = (BC - 1 + BATCH_ + BC - 1) // BC
OVC = 2 * G2 * BC


def _body(s_ref, ovl_ref, cache_ref, out_ref):
    jc = pl.program_id(1)
    idx = s_ref[1]
    e1 = s_ref[2]
    e2 = s_ref[3]

    near = ((jc >= s_ref[0]) & (jc < s_ref[0] + G2)) | (jc < G2)

    @pl.when(jnp.logical_not(near))
    def _copy():
        out_ref[...] = cache_ref[...]

    @pl.when(near)
    def _merge():
        c = jc * BC + jax.lax.broadcasted_iota(jnp.int32, (BF, BC), 1)
        mask = ((c >= idx) & (c < e1)) | (c < e2)
        out_ref[...] = jnp.where(mask, ovl_ref[...], cache_ref[...])


def kernel(activations, cache, n_valid, index):
    max_samples = cache.shape[0]
    batch = activations.shape[0]
    index = jnp.asarray(index) % max_samples
    new_n_valid = jnp.minimum(jnp.asarray(n_valid) + batch, max_samples)
    new_index = (index + batch) % max_samples

    q1 = (index // BC).astype(jnp.int32)
    im1 = (index % BC).astype(jnp.int32)
    e1 = jnp.minimum(index + batch, max_samples).astype(jnp.int32)
    e2 = (index + batch - max_samples).astype(jnp.int32)
    sp = jnp.stack([q1, index.astype(jnp.int32), e1, e2])

    cache_t = jax.lax.bitcast_convert_type(cache, jnp.bfloat16).T
    acts_t = jax.lax.bitcast_convert_type(
        activations.astype(jnp.float16), jnp.bfloat16).T

    ovl1 = jax.lax.dynamic_update_slice(
        jnp.zeros((N_DIM_, G2 * BC), jnp.bfloat16), acts_t, (0, im1))
    delta = jnp.where(e2 > 0, max_samples - index, batch)
    ovl2 = jax.lax.dynamic_slice(
        jnp.concatenate(
            [acts_t, jnp.zeros((N_DIM_, G2 * BC), jnp.bfloat16)], axis=1),
        (0, delta), (N_DIM_, G2 * BC))
    ovl = jnp.concatenate([ovl1, ovl2], axis=1)

    def ovl_map(f, jc, s):
        in_w1 = (jc >= s[0]) & (jc < s[0] + G2)
        return (f, jnp.where(in_w1, jc - s[0],
                             jnp.where(jc < G2, G2 + jc, 0)))

    grid_spec = pltpu.PrefetchScalarGridSpec(
        num_scalar_prefetch=1,
        grid=(NBF, NBC),
        in_specs=[
            pl.BlockSpec((BF, BC), ovl_map),
            pl.BlockSpec((BF, BC), lambda f, jc, s: (f, jc)),
        ],
        out_specs=pl.BlockSpec((BF, BC), lambda f, jc, s: (f, jc)),
    )

    out_t = pl.pallas_call(
        _body,
        grid_spec=grid_spec,
        out_shape=jax.ShapeDtypeStruct((N_DIM_, max_samples), jnp.bfloat16),
    )(sp, ovl, cache_t)

    new_cache = jax.lax.bitcast_convert_type(out_t.T, jnp.float16)
    return (new_cache, new_n_valid, new_index)
